# baseline (speedup 1.0000x reference)
import functools

import jax
import jax.numpy as jnp
from jax import lax
from jax.experimental import pallas as pl
from jax.experimental.pallas import tpu as pltpu
from jax.experimental.pallas import tpu_sc as plsc

MAX_REL = 64
DIM = 64
VOCAB = 2 * MAX_REL + 1
LEN = 2048
G_ROWS = 2 * LEN
SHIFT = LEN - MAX_REL - 1

NUM_CORES = 2
NUM_SUBCORES = 16
NUM_WORKERS = NUM_CORES * NUM_SUBCORES
GROWS_PER_WORKER = G_ROWS // NUM_WORKERS
LANES = 16


def _sc_gather_body(w_hbm, g_hbm, idx_v, rows_v, sem):
    c = lax.axis_index("c")
    s = lax.axis_index("s")
    wid = s * NUM_CORES + c
    t0 = wid * GROWS_PER_WORKER
    for k in range(GROWS_PER_WORKER // LANES):
        t = lax.iota(jnp.int32, LANES) + (t0 + k * LANES - SHIFT)
        idx_v[pl.ds(k * LANES, LANES)] = jnp.clip(t, 0, VOCAB - 1)
    pltpu.async_copy(w_hbm.at[idx_v], rows_v, sem).wait()
    pltpu.sync_copy(rows_v, g_hbm.at[pl.ds(t0, GROWS_PER_WORKER)])


@functools.cache
def _sc_gather():
    return pl.kernel(
        _sc_gather_body,
        out_type=jax.ShapeDtypeStruct((G_ROWS, 2 * DIM), jnp.float32),
        mesh=plsc.VectorSubcoreMesh(
            core_axis_name="c", subcore_axis_name="s", num_cores=NUM_CORES,
            num_subcores=NUM_SUBCORES),
        scratch_types=[
            pltpu.VMEM((GROWS_PER_WORKER,), jnp.int32),
            pltpu.VMEM((GROWS_PER_WORKER, 2 * DIM), jnp.float32),
            pltpu.SemaphoreType.DMA,
        ],
    )


BI = 8


def _bcast_t(gt_ref, out_ref):
    b = pl.program_id(0)
    gt = gt_ref[...]
    for ii in range(BI):
        s = (LEN - 1 - ii) - BI * b
        amount = lax.rem(G_ROWS - s, G_ROWS)
        out_ref[ii] = pltpu.roll(gt, amount, axis=1)[:, :LEN]


def _tc_broadcast_t(gt):
    return pl.pallas_call(
        _bcast_t,
        grid=(LEN // BI,),
        in_specs=[pl.BlockSpec((DIM, G_ROWS), lambda b: (0, 0))],
        out_specs=pl.BlockSpec((BI, DIM, LEN), lambda b: (b, 0, 0)),
        out_shape=jax.ShapeDtypeStruct((LEN, DIM, LEN), jnp.float32),
    )(gt)


def kernel(len_in, len_out, W):
    del len_in, len_out
    w_pad = jnp.pad(W, ((0, 0), (0, DIM)))
    g = _sc_gather()(w_pad)
    gt = jnp.transpose(g[:, :DIM])
    p = _tc_broadcast_t(gt)
    return jnp.transpose(p, (0, 2, 1))


# speedup vs baseline: 1.5692x; 1.5692x over previous
import functools

import jax
import jax.numpy as jnp
from jax import lax
from jax.experimental import pallas as pl
from jax.experimental.pallas import tpu as pltpu
from jax.experimental.pallas import tpu_sc as plsc

MAX_REL = 64
DIM = 64
VOCAB = 2 * MAX_REL + 1
LEN = 2048
G_ROWS = 2 * LEN
SHIFT = LEN - MAX_REL - 1

NUM_CORES = 2
NUM_SUBCORES = 16
NUM_WORKERS = NUM_CORES * NUM_SUBCORES
GROWS_PER_WORKER = G_ROWS // NUM_WORKERS
LANES = 16


def _sc_gather_body(w_hbm, g_hbm, idx_v, rows_v, sem):
    c = lax.axis_index("c")
    s = lax.axis_index("s")
    wid = s * NUM_CORES + c
    t0 = wid * GROWS_PER_WORKER
    for k in range(GROWS_PER_WORKER // LANES):
        t = lax.iota(jnp.int32, LANES) + (t0 + k * LANES - SHIFT)
        idx_v[pl.ds(k * LANES, LANES)] = jnp.clip(t, 0, VOCAB - 1)
    pltpu.async_copy(w_hbm.at[idx_v], rows_v, sem).wait()
    pltpu.sync_copy(rows_v, g_hbm.at[pl.ds(t0, GROWS_PER_WORKER)])


@functools.cache
def _sc_gather():
    return pl.kernel(
        _sc_gather_body,
        out_type=jax.ShapeDtypeStruct((G_ROWS, 2 * DIM), jnp.float32),
        mesh=plsc.VectorSubcoreMesh(
            core_axis_name="c", subcore_axis_name="s", num_cores=NUM_CORES,
            num_subcores=NUM_SUBCORES),
        scratch_types=[
            pltpu.VMEM((GROWS_PER_WORKER,), jnp.int32),
            pltpu.VMEM((GROWS_PER_WORKER, 2 * DIM), jnp.float32),
            pltpu.SemaphoreType.DMA,
        ],
    )


BI = 8


def _bcast_t(gt_ref, out_ref):
    b = pl.program_id(0)
    for ii in range(BI):
        s = (LEN - 1 - ii) - BI * b
        q = pl.multiple_of((s // 128) * 128, 128)
        r = s - q
        win = gt_ref[:, pl.ds(q, LEN + 128)]
        amount = lax.rem((LEN + 128) - r, LEN + 128)
        out_ref[ii] = pltpu.roll(win, amount, axis=1)[:, :LEN]


def _tc_broadcast_t(gt):
    return pl.pallas_call(
        _bcast_t,
        grid=(LEN // BI,),
        in_specs=[pl.BlockSpec((DIM, G_ROWS), lambda b: (0, 0))],
        out_specs=pl.BlockSpec((BI, DIM, LEN), lambda b: (b, 0, 0)),
        out_shape=jax.ShapeDtypeStruct((LEN, DIM, LEN), jnp.float32),
    )(gt)


def kernel(len_in, len_out, W):
    del len_in, len_out
    w_pad = jnp.pad(W, ((0, 0), (0, DIM)))
    g = _sc_gather()(w_pad)
    gt = jnp.transpose(g[:, :DIM])
    p = _tc_broadcast_t(gt)
    return jnp.transpose(p, (0, 2, 1))


# speedup vs baseline: 1.7044x; 1.0861x over previous
import functools

import jax
import jax.numpy as jnp
from jax import lax
from jax.experimental import pallas as pl
from jax.experimental.pallas import tpu as pltpu
from jax.experimental.pallas import tpu_sc as plsc

MAX_REL = 64
DIM = 64
VOCAB = 2 * MAX_REL + 1
LEN = 2048
G_ROWS = 2 * LEN
SHIFT = LEN - MAX_REL - 1

NUM_CORES = 2
NUM_SUBCORES = 16
NUM_WORKERS = NUM_CORES * NUM_SUBCORES
GROWS_PER_WORKER = G_ROWS // NUM_WORKERS
LANES = 16


def _sc_gather_body(w_hbm, g_hbm, idx_v, rows_v, sem):
    c = lax.axis_index("c")
    s = lax.axis_index("s")
    wid = s * NUM_CORES + c
    t0 = wid * GROWS_PER_WORKER
    for k in range(GROWS_PER_WORKER // LANES):
        t = lax.iota(jnp.int32, LANES) + (t0 + k * LANES - SHIFT)
        idx_v[pl.ds(k * LANES, LANES)] = jnp.clip(t, 0, VOCAB - 1)
    pltpu.async_copy(w_hbm.at[idx_v], rows_v, sem).wait()
    pltpu.sync_copy(rows_v, g_hbm.at[pl.ds(t0, GROWS_PER_WORKER)])


@functools.cache
def _sc_gather():
    return pl.kernel(
        _sc_gather_body,
        out_type=jax.ShapeDtypeStruct((G_ROWS, 2 * DIM), jnp.float32),
        mesh=plsc.VectorSubcoreMesh(
            core_axis_name="c", subcore_axis_name="s", num_cores=NUM_CORES,
            num_subcores=NUM_SUBCORES),
        scratch_types=[
            pltpu.VMEM((GROWS_PER_WORKER,), jnp.int32),
            pltpu.VMEM((GROWS_PER_WORKER, 2 * DIM), jnp.float32),
            pltpu.SemaphoreType.DMA,
        ],
    )


BI = 16


def _bcast_t(gt_ref, out_ref):
    b = pl.program_id(0)
    for ii in range(BI):
        s = (LEN - 1 - ii) - BI * b
        q = pl.multiple_of((s // 128) * 128, 128)
        r = s - q
        win = gt_ref[:, pl.ds(q, LEN + 128)]
        amount = lax.rem((LEN + 128) - r, LEN + 128)
        out_ref[ii] = pltpu.roll(win, amount, axis=1)[:, :LEN]


def _tc_broadcast_t(gt):
    return pl.pallas_call(
        _bcast_t,
        grid=(LEN // BI,),
        in_specs=[pl.BlockSpec((DIM, G_ROWS), lambda b: (0, 0))],
        out_specs=pl.BlockSpec((BI, DIM, LEN), lambda b: (b, 0, 0)),
        out_shape=jax.ShapeDtypeStruct((LEN, DIM, LEN), jnp.float32),
    )(gt)


def kernel(len_in, len_out, W):
    del len_in, len_out
    w_pad = jnp.pad(W, ((0, 0), (0, DIM)))
    g = _sc_gather()(w_pad)
    gt = jnp.transpose(g[:, :DIM])
    p = _tc_broadcast_t(gt)
    return jnp.transpose(p, (0, 2, 1))


# speedup vs baseline: 1.7524x; 1.0282x over previous
import functools

import jax
import jax.numpy as jnp
from jax import lax
from jax.experimental import pallas as pl
from jax.experimental.pallas import tpu as pltpu
from jax.experimental.pallas import tpu_sc as plsc

MAX_REL = 64
DIM = 64
VOCAB = 2 * MAX_REL + 1
LEN = 2048
G_ROWS = 2 * LEN
SHIFT = LEN - MAX_REL - 1

NUM_CORES = 2
NUM_SUBCORES = 16
NUM_WORKERS = NUM_CORES * NUM_SUBCORES
GROWS_PER_WORKER = G_ROWS // NUM_WORKERS
LANES = 16


def _sc_gather_body(w_hbm, g_hbm, idx_v, rows_v, sem):
    c = lax.axis_index("c")
    s = lax.axis_index("s")
    wid = s * NUM_CORES + c
    t0 = wid * GROWS_PER_WORKER
    for k in range(GROWS_PER_WORKER // LANES):
        t = lax.iota(jnp.int32, LANES) + (t0 + k * LANES - SHIFT)
        idx_v[pl.ds(k * LANES, LANES)] = jnp.clip(t, 0, VOCAB - 1)
    pltpu.async_copy(w_hbm.at[idx_v], rows_v, sem).wait()
    pltpu.sync_copy(rows_v, g_hbm.at[pl.ds(t0, GROWS_PER_WORKER)])


@functools.cache
def _sc_gather():
    return pl.kernel(
        _sc_gather_body,
        out_type=jax.ShapeDtypeStruct((G_ROWS, 2 * DIM), jnp.float32),
        mesh=plsc.VectorSubcoreMesh(
            core_axis_name="c", subcore_axis_name="s", num_cores=NUM_CORES,
            num_subcores=NUM_SUBCORES),
        scratch_types=[
            pltpu.VMEM((GROWS_PER_WORKER,), jnp.int32),
            pltpu.VMEM((GROWS_PER_WORKER, 2 * DIM), jnp.float32),
            pltpu.SemaphoreType.DMA,
        ],
    )


BI = 32


def _bcast_t(gt_ref, out_ref):
    b = pl.program_id(0)
    for ii in range(BI):
        s = (LEN - 1 - ii) - BI * b
        q = pl.multiple_of((s // 128) * 128, 128)
        r = s - q
        win = gt_ref[:, pl.ds(q, LEN + 128)]
        amount = lax.rem((LEN + 128) - r, LEN + 128)
        out_ref[ii] = pltpu.roll(win, amount, axis=1)[:, :LEN]


def _tc_broadcast_t(gt):
    return pl.pallas_call(
        _bcast_t,
        grid=(LEN // BI,),
        in_specs=[pl.BlockSpec((DIM, G_ROWS), lambda b: (0, 0))],
        out_specs=pl.BlockSpec((BI, DIM, LEN), lambda b: (b, 0, 0)),
        out_shape=jax.ShapeDtypeStruct((LEN, DIM, LEN), jnp.float32),
    )(gt)


def kernel(len_in, len_out, W):
    del len_in, len_out
    w_pad = jnp.pad(W, ((0, 0), (0, DIM)))
    g = _sc_gather()(w_pad)
    gt = jnp.transpose(g[:, :DIM])
    p = _tc_broadcast_t(gt)
    return jnp.transpose(p, (0, 2, 1))
